# Initial kernel scaffold; baseline (speedup 1.0000x reference)
import jax
import jax.numpy as jnp
from jax.experimental import pallas as pl


def kernel(x, table):
    raise NotImplementedError("write your pallas kernel here")


# speedup vs baseline: 1.5533x; 1.5533x over previous
import functools

import jax
import jax.numpy as jnp
from jax import lax
from jax.experimental import pallas as pl
from jax.experimental.pallas import tpu as pltpu
from jax.experimental.pallas import tpu_sc as plsc

_VOCAB = 100000
_D = 1024
_BATCH = 4
_SEQ = 2048
_NB = _BATCH * _SEQ

_info = plsc.get_sparse_core_info()
_NC = _info.num_cores
_NS = _info.num_subcores
_NW = _NC * _NS
_BPW = _NB // _NW
_C = 32
_NCHUNK = _BPW // _C

_mesh = plsc.VectorSubcoreMesh(core_axis_name="c", subcore_axis_name="s")


@functools.partial(
    pl.kernel,
    mesh=_mesh,
    out_type=jax.ShapeDtypeStruct((_NB, _D), jnp.float32),
    scratch_types=[
        pltpu.VMEM((_NCHUNK, _C), jnp.int32),
        pltpu.VMEM((2, _C, _D), jnp.float32),
        pltpu.SemaphoreType.DMA,
        pltpu.SemaphoreType.DMA,
    ],
)
def _embed_sc(x_hbm, table_hbm, out_hbm, idx_v, buf_v, gsem, ssem):
    wid = lax.axis_index("s") * _NC + lax.axis_index("c")
    base = wid * _BPW

    pltpu.sync_copy(x_hbm.at[wid], idx_v)

    pltpu.async_copy(table_hbm.at[idx_v.at[0]], buf_v.at[0], gsem)

    for j in range(_NCHUNK):
        b = j % 2
        if j >= 1:
            pltpu.make_async_copy(
                buf_v.at[(j - 1) % 2],
                out_hbm.at[pl.ds(base + (j - 1) * _C, _C)],
                ssem,
            ).wait()
        if j + 1 < _NCHUNK:
            pltpu.async_copy(
                table_hbm.at[idx_v.at[j + 1]], buf_v.at[(j + 1) % 2], gsem
            )
        pltpu.make_async_copy(
            table_hbm.at[idx_v.at[j]], buf_v.at[b], gsem
        ).wait()
        pltpu.async_copy(
            buf_v.at[b], out_hbm.at[pl.ds(base + j * _C, _C)], ssem
        )

    pltpu.make_async_copy(
        buf_v.at[(_NCHUNK - 1) % 2],
        out_hbm.at[pl.ds(base + (_NCHUNK - 1) * _C, _C)],
        ssem,
    ).wait()


def kernel(x, table):
    x3 = x.reshape(_NW, _NCHUNK, _C)
    out = _embed_sc(x3, table)
    return out.reshape(_BATCH, _SEQ, _D)


# speedup vs baseline: 1.5629x; 1.0062x over previous
import functools

import jax
import jax.numpy as jnp
from jax import lax
from jax.experimental import pallas as pl
from jax.experimental.pallas import tpu as pltpu
from jax.experimental.pallas import tpu_sc as plsc

_VOCAB = 100000
_D = 1024
_BATCH = 4
_SEQ = 2048
_NB = _BATCH * _SEQ

_info = plsc.get_sparse_core_info()
_NC = _info.num_cores
_NS = _info.num_subcores
_NW = _NC * _NS
_BPW = _NB // _NW
_C = 32
_NCHUNK = _BPW // _C

_mesh = plsc.VectorSubcoreMesh(core_axis_name="c", subcore_axis_name="s")


_NBUF = 3


@functools.partial(
    pl.kernel,
    mesh=_mesh,
    out_type=jax.ShapeDtypeStruct((_NB, _D), jnp.float32),
    scratch_types=[
        pltpu.VMEM((_NCHUNK, _C), jnp.int32),
        pltpu.VMEM((_NBUF, _C, _D), jnp.float32),
        pltpu.SemaphoreType.DMA,
        pltpu.SemaphoreType.DMA,
    ],
)
def _embed_sc(x_hbm, table_hbm, out_hbm, idx_v, buf_v, gsem, ssem):
    wid = lax.axis_index("s") * _NC + lax.axis_index("c")
    base = wid * _BPW

    def gather(j):
        pltpu.async_copy(table_hbm.at[idx_v.at[j]], buf_v.at[j % _NBUF], gsem)

    def gather_wait(j):
        pltpu.make_async_copy(
            table_hbm.at[idx_v.at[j]], buf_v.at[j % _NBUF], gsem
        ).wait()

    def store(j):
        pltpu.async_copy(
            buf_v.at[j % _NBUF], out_hbm.at[pl.ds(base + j * _C, _C)], ssem
        )

    def store_wait(j):
        pltpu.make_async_copy(
            buf_v.at[j % _NBUF], out_hbm.at[pl.ds(base + j * _C, _C)], ssem
        ).wait()

    pltpu.sync_copy(x_hbm.at[wid], idx_v)

    gather(0)
    gather(1)

    for j in range(_NCHUNK):
        gather_wait(j)
        store(j)
        if j + 2 < _NCHUNK:
            if j >= 1:
                store_wait(j - 1)
            gather(j + 2)

    for j in range(_NCHUNK - _NBUF, _NCHUNK):
        store_wait(j)


def kernel(x, table):
    x3 = x.reshape(_NW, _NCHUNK, _C)
    out = _embed_sc(x3, table)
    return out.reshape(_BATCH, _SEQ, _D)
